# Initial kernel scaffold; baseline (speedup 1.0000x reference)
import jax
import jax.numpy as jnp
from jax.experimental import pallas as pl


def kernel(x, conv_w, temb_w, W_qk, W_v, W_o, b_o, ln1_g, ln1_b, ln2_g, ln2_b, ff1_w, ff1_b, ff2_w, ff2_b, enc_norm_g, enc_norm_b, proj_w, proj_b):
    raise NotImplementedError("write your pallas kernel here")


# speedup vs baseline: 1.0087x; 1.0087x over previous
import functools

import jax
import jax.numpy as jnp
import numpy as np
from jax.experimental import pallas as pl

T, N, NX, NY = 2048, 2, 32, 32
D = 768
H = 4
DH = D // H
DFF = 4 * D
L = 2
BUCKET = 4
NH = 4


def _pos_emb(seq, d):
    pos = np.arange(seq)[:, None].astype(np.float64)
    i = np.arange(d)[None, :].astype(np.float64)
    angle = pos / np.power(10000.0, (2.0 * (i // 2)) / d)
    pe = np.zeros((seq, d), dtype=np.float32)
    pe[:, 0::2] = np.sin(angle[:, 0::2])
    pe[:, 1::2] = np.cos(angle[:, 1::2])
    return jnp.asarray(pe)


def _ln(x, g, b):
    m = x.mean(-1, keepdims=True)
    v = x.var(-1, keepdims=True)
    return (x - m) / jnp.sqrt(v + 1e-5) * g + b


def _lsh_attn(x, Wqk, Wv, Wo, bo, rot):
    B, S, Dm = x.shape
    qk = (x @ Wqk).reshape(B, S, H, DH).transpose(0, 2, 1, 3)
    v = (x @ Wv).reshape(B, S, H, DH).transpose(0, 2, 1, 3)
    kn = qk / (jnp.linalg.norm(qk, axis=-1, keepdims=True) + 1e-8)
    pos = jnp.arange(S)
    c = BUCKET
    nc = S // c
    outs, lses = [], []
    for r in range(NH):
        rotated = jnp.einsum('bhsd,dk->bhsk', qk, rot[r])
        buckets = jnp.argmax(jnp.concatenate([rotated, -rotated], -1), -1)
        comb = buckets * S + pos[None, None, :]
        sidx = jnp.argsort(comb, axis=-1)
        uidx = jnp.argsort(sidx, axis=-1)
        sq = jnp.take_along_axis(qk, sidx[..., None], axis=2)
        sk = jnp.take_along_axis(kn, sidx[..., None], axis=2)
        sv = jnp.take_along_axis(v, sidx[..., None], axis=2)
        sp = jnp.take_along_axis(jnp.broadcast_to(pos[None, None, :], comb.shape), sidx, axis=-1)
        qc = sq.reshape(B, H, nc, c, DH)
        kc = sk.reshape(B, H, nc, c, DH)
        vc = sv.reshape(B, H, nc, c, DH)
        pc = sp.reshape(B, H, nc, c)
        kk = jnp.concatenate([kc, jnp.roll(kc, 1, axis=2)], axis=3)
        vv = jnp.concatenate([vc, jnp.roll(vc, 1, axis=2)], axis=3)
        pk = jnp.concatenate([pc, jnp.roll(pc, 1, axis=2)], axis=3)
        dots = jnp.einsum('bhncd,bhnkd->bhnck', qc, kk) / (DH ** 0.5)
        self_mask = pc[..., :, None] == pk[..., None, :]
        dots = jnp.where(self_mask, -1e5, dots)
        lse = jax.nn.logsumexp(dots, axis=-1)
        attn = jnp.exp(dots - lse[..., None])
        oc = jnp.einsum('bhnck,bhnkd->bhncd', attn, vv)
        os_ = oc.reshape(B, H, S, DH)
        ls_ = lse.reshape(B, H, S)
        outs.append(jnp.take_along_axis(os_, uidx[..., None], axis=2))
        lses.append(jnp.take_along_axis(ls_, uidx, axis=2))
    outs = jnp.stack(outs, 0)
    lses = jnp.stack(lses, 0)
    w = jax.nn.softmax(lses, axis=0)[..., None]
    o = (w * outs).sum(0)
    o = o.transpose(0, 2, 1, 3).reshape(B, S, Dm)
    return o @ Wo + bo


def _final_proj_kernel(h_ref, g_ref, b_ref, w_ref, pb_ref, o_ref):
    h = h_ref[0]
    m = jnp.mean(h, axis=-1, keepdims=True)
    v = jnp.mean((h - m) * (h - m), axis=-1, keepdims=True)
    xn = (h - m) / jnp.sqrt(v + 1e-5) * g_ref[0, 0] + b_ref[0, 0]
    o_ref[0] = jnp.dot(xn, w_ref[...], preferred_element_type=jnp.float32) + pb_ref[0, 0]


def _final_proj(h, g, b, w, pb):
    B, S, Dm = h.shape
    SB = 512
    grid = (B, S // SB)
    return pl.pallas_call(
        _final_proj_kernel,
        grid=grid,
        in_specs=[
            pl.BlockSpec((1, SB, Dm), lambda i, j: (i, j, 0)),
            pl.BlockSpec((1, 1, Dm), lambda i, j: (0, 0, 0)),
            pl.BlockSpec((1, 1, Dm), lambda i, j: (0, 0, 0)),
            pl.BlockSpec((Dm, NY), lambda i, j: (0, 0)),
            pl.BlockSpec((1, 1, NY), lambda i, j: (0, 0, 0)),
        ],
        out_specs=pl.BlockSpec((1, SB, NY), lambda i, j: (i, j, 0)),
        out_shape=jax.ShapeDtypeStruct((B, S, NY), jnp.float32),
    )(h, g.reshape(1, 1, Dm), b.reshape(1, 1, Dm), w, pb.reshape(1, 1, NY))


def kernel(x, conv_w, temb_w, W_qk, W_v, W_o, b_o, ln1_g, ln1_b, ln2_g, ln2_b,
           ff1_w, ff1_b, ff2_w, ff2_b, enc_norm_g, enc_norm_b, proj_w, proj_b):
    rot = jax.random.normal(jax.random.key(42), (NH, DH, (T // BUCKET) // 2), dtype=jnp.float32)
    x_enc = x.transpose(1, 0, 2)
    xp = jnp.concatenate([x_enc[:, -1:], x_enc, x_enc[:, :1]], axis=1)
    tok = sum(jnp.einsum('btf,fd->btd', xp[:, k:k + T], conv_w[k]) for k in range(3))
    h = tok + _pos_emb(T, D)[None]
    for l in range(L):
        a = _lsh_attn(h, W_qk[l], W_v[l], W_o[l], b_o[l], rot)
        x1 = h + a
        xn = _ln(x1, ln1_g[l], ln1_b[l])
        y = jax.nn.gelu(xn @ ff1_w[l] + ff1_b[l], approximate=False)
        y = y @ ff2_w[l] + ff2_b[l]
        h = _ln(xn + y, ln2_g[l], ln2_b[l])
    out = _final_proj(h, enc_norm_g, enc_norm_b, proj_w, proj_b)
    return out.transpose(1, 0, 2)


# speedup vs baseline: 2.5410x; 2.5192x over previous
import functools

import jax
import jax.numpy as jnp
import numpy as np
from jax.experimental import pallas as pl

T, N, NX, NY = 2048, 2, 32, 32
D = 768
H = 4
DH = D // H
DFF = 4 * D
L = 2
BUCKET = 4
NH = 4
B = N
S = T
NROT = (T // BUCKET) // 2
W = NH * B * H
KB = 256
NBLK = S // KB
SCALE = 1.0 / (DH ** 0.5)
NEG = -1e30


def _pos_emb(seq, d):
    pos = np.arange(seq)[:, None].astype(np.float64)
    i = np.arange(d)[None, :].astype(np.float64)
    angle = pos / np.power(10000.0, (2.0 * (i // 2)) / d)
    pe = np.zeros((seq, d), dtype=np.float32)
    pe[:, 0::2] = np.sin(angle[:, 0::2])
    pe[:, 1::2] = np.cos(angle[:, 1::2])
    return jnp.asarray(pe)


def _ln_rows(x, g, b):
    m = jnp.mean(x, axis=-1, keepdims=True)
    v = jnp.mean((x - m) * (x - m), axis=-1, keepdims=True)
    return (x - m) / jnp.sqrt(v + 1e-5) * g + b


def _front_kernel(xp_ref, cw_ref, pe_ref, o_ref):
    xp = xp_ref[0]
    t0 = jnp.dot(xp[0:S, :], cw_ref[0], preferred_element_type=jnp.float32)
    t1 = jnp.dot(xp[1:S + 1, :], cw_ref[1], preferred_element_type=jnp.float32)
    t2 = jnp.dot(xp[2:S + 2, :], cw_ref[2], preferred_element_type=jnp.float32)
    o_ref[0] = ((t0 + t1) + t2) + pe_ref[...]


def _front(xp, cw_cat, pe):
    return pl.pallas_call(
        _front_kernel,
        grid=(B,),
        in_specs=[
            pl.BlockSpec((1, S + 2, NX), lambda i: (i, 0, 0)),
            pl.BlockSpec((3, NX, D), lambda i: (0, 0, 0)),
            pl.BlockSpec((S, D), lambda i: (0, 0)),
        ],
        out_specs=pl.BlockSpec((1, S, D), lambda i: (i, 0, 0)),
        out_shape=jax.ShapeDtypeStruct((B, S, D), jnp.float32),
    )(xp, cw_cat, pe)


def _qkv_kernel(h_ref, wqk_ref, wv_ref, rot_ref, qkv_ref, bk_ref):
    h = h_ref[0]
    qk = jnp.dot(h, wqk_ref[...], preferred_element_type=jnp.float32)
    v = jnp.dot(h, wv_ref[...], preferred_element_type=jnp.float32)
    for hh in range(H):
        qk_h = qk[:, hh * DH:(hh + 1) * DH]
        qkv_ref[0, hh, :, 0:DH] = qk_h
        qkv_ref[0, hh, :, DH:2 * DH] = v[:, hh * DH:(hh + 1) * DH]
        for r in range(NH):
            rot = jnp.dot(qk_h, rot_ref[r], preferred_element_type=jnp.float32)
            m1 = jnp.max(rot, axis=-1, keepdims=True)
            i1 = jnp.argmax(rot, axis=-1, keepdims=True)
            m2 = -jnp.min(rot, axis=-1, keepdims=True)
            i2 = jnp.argmin(rot, axis=-1, keepdims=True)
            bk_ref[r, 0, hh] = jnp.where(m1 >= m2, i1, i2 + NROT).astype(jnp.int32)


def _qkv_buckets(h, wqk, wv, rot):
    SB = 512
    return pl.pallas_call(
        _qkv_kernel,
        grid=(B, S // SB),
        in_specs=[
            pl.BlockSpec((1, SB, D), lambda i, j: (i, j, 0)),
            pl.BlockSpec((D, D), lambda i, j: (0, 0)),
            pl.BlockSpec((D, D), lambda i, j: (0, 0)),
            pl.BlockSpec((NH, DH, NROT), lambda i, j: (0, 0, 0)),
        ],
        out_specs=[
            pl.BlockSpec((1, H, SB, 2 * DH), lambda i, j: (i, 0, j, 0)),
            pl.BlockSpec((NH, 1, H, SB, 1), lambda i, j: (0, i, 0, j, 0)),
        ],
        out_shape=[
            jax.ShapeDtypeStruct((B, H, S, 2 * DH), jnp.float32),
            jax.ShapeDtypeStruct((NH, B, H, S, 1), jnp.int32),
        ],
    )(h, wqk, wv, rot)


def _attn_kernel(sqkv_ref, tail_ref, o_ref, lse_ref):
    blk = sqkv_ref[0]
    q = blk[:, 0:DH]
    k = blk[:, 0:DH]
    vv = blk[:, DH:2 * DH]
    kn = k / (jnp.sqrt(jnp.sum(k * k, axis=-1, keepdims=True)) + 1e-8)
    tail = tail_ref[0]
    kt = tail[:, 0:DH]
    vt = tail[:, DH:2 * DH]
    knt = kt / (jnp.sqrt(jnp.sum(kt * kt, axis=-1, keepdims=True)) + 1e-8)

    dots = jax.lax.dot_general(q, kn, (((1,), (1,)), ((), ())),
                               preferred_element_type=jnp.float32) / (DH ** 0.5)
    dots_t = jax.lax.dot_general(q, knt, (((1,), (1,)), ((), ())),
                                 preferred_element_type=jnp.float32) / (DH ** 0.5)

    ri = jax.lax.broadcasted_iota(jnp.int32, (KB, KB), 0)
    ci = jax.lax.broadcasted_iota(jnp.int32, (KB, KB), 1)
    band = ((ri // BUCKET) == (ci // BUCKET)) | ((ri // BUCKET) == (ci // BUCKET) + 1)
    dots = jnp.where(ri == ci, -1e5, dots)
    dots = jnp.where(band, dots, NEG)

    rows_t = jax.lax.broadcasted_iota(jnp.int32, (KB, BUCKET), 0)
    dots_t = jnp.where(rows_t < BUCKET, dots_t, NEG)

    m = jnp.maximum(jnp.max(dots, axis=-1, keepdims=True),
                    jnp.max(dots_t, axis=-1, keepdims=True))
    e = jnp.exp(dots - m)
    et = jnp.exp(dots_t - m)
    denom = jnp.sum(e, axis=-1, keepdims=True) + jnp.sum(et, axis=-1, keepdims=True)
    lse = m + jnp.log(denom)
    a = jnp.exp(dots - lse)
    at = jnp.exp(dots_t - lse)
    o = jnp.dot(a, vv, preferred_element_type=jnp.float32)
    o = o + jnp.dot(at, vt, preferred_element_type=jnp.float32)
    o_ref[0] = o
    lse_ref[0] = lse


def _attention(sqkv, tails):
    return pl.pallas_call(
        _attn_kernel,
        grid=(W, NBLK),
        in_specs=[
            pl.BlockSpec((1, KB, 2 * DH), lambda w, j: (w, j, 0)),
            pl.BlockSpec((1, BUCKET, 2 * DH), lambda w, j: (w * NBLK + j, 0, 0)),
        ],
        out_specs=[
            pl.BlockSpec((1, KB, DH), lambda w, j: (w, j, 0)),
            pl.BlockSpec((1, KB, 1), lambda w, j: (w, j, 0)),
        ],
        out_shape=[
            jax.ShapeDtypeStruct((W, S, DH), jnp.float32),
            jax.ShapeDtypeStruct((W, S, 1), jnp.float32),
        ],
    )(sqkv, tails)


def _comb_kernel(outs_ref, lses_ref, h_ref, wo_ref, bo_ref, g_ref, b_ref, xn_ref):
    ls = [lses_ref[r, 0] for r in range(NH)]
    m = ls[0]
    for r in range(1, NH):
        m = jnp.maximum(m, ls[r])
    es = [jnp.exp(ls[r] - m) for r in range(NH)]
    den = es[0]
    for r in range(1, NH):
        den = den + es[r]
    y = jnp.zeros_like(h_ref[0])
    for hh in range(H):
        oh = None
        for r in range(NH):
            wgt = es[r][:, hh:hh + 1] / den[:, hh:hh + 1]
            contrib = wgt * outs_ref[r, 0, hh]
            oh = contrib if oh is None else oh + contrib
        y = y + jnp.dot(oh, wo_ref[hh * DH:(hh + 1) * DH, :],
                        preferred_element_type=jnp.float32)
    x1 = h_ref[0] + y + bo_ref[0, 0]
    xn_ref[0] = _ln_rows(x1, g_ref[0, 0], b_ref[0, 0])


def _combine(outs, lses, h, wo, bo, g, b):
    SB = 512
    return pl.pallas_call(
        _comb_kernel,
        grid=(B, S // SB),
        in_specs=[
            pl.BlockSpec((NH, 1, H, SB, DH), lambda i, j: (0, i, 0, j, 0)),
            pl.BlockSpec((NH, 1, SB, H), lambda i, j: (0, i, j, 0)),
            pl.BlockSpec((1, SB, D), lambda i, j: (i, j, 0)),
            pl.BlockSpec((D, D), lambda i, j: (0, 0)),
            pl.BlockSpec((1, 1, D), lambda i, j: (0, 0, 0)),
            pl.BlockSpec((1, 1, D), lambda i, j: (0, 0, 0)),
            pl.BlockSpec((1, 1, D), lambda i, j: (0, 0, 0)),
        ],
        out_specs=pl.BlockSpec((1, SB, D), lambda i, j: (i, j, 0)),
        out_shape=jax.ShapeDtypeStruct((B, S, D), jnp.float32),
    )(outs, lses, h, wo, bo.reshape(1, 1, D), g.reshape(1, 1, D), b.reshape(1, 1, D))


def _ffn_kernel(xn_ref, w1_ref, b1_ref, w2_ref, b2_ref, g_ref, b_ref, o_ref):
    xn = xn_ref[0]
    y1 = jnp.dot(xn, w1_ref[...], preferred_element_type=jnp.float32) + b1_ref[0, 0]
    y1 = 0.5 * y1 * (1.0 + jax.lax.erf(y1 * (2.0 ** -0.5)))
    y = jnp.dot(y1, w2_ref[...], preferred_element_type=jnp.float32) + b2_ref[0, 0]
    o_ref[0] = _ln_rows(xn + y, g_ref[0, 0], b_ref[0, 0])


def _ffn(xn, w1, b1, w2, b2, g, b):
    SB = 512
    return pl.pallas_call(
        _ffn_kernel,
        grid=(B, S // SB),
        in_specs=[
            pl.BlockSpec((1, SB, D), lambda i, j: (i, j, 0)),
            pl.BlockSpec((D, DFF), lambda i, j: (0, 0)),
            pl.BlockSpec((1, 1, DFF), lambda i, j: (0, 0, 0)),
            pl.BlockSpec((DFF, D), lambda i, j: (0, 0)),
            pl.BlockSpec((1, 1, D), lambda i, j: (0, 0, 0)),
            pl.BlockSpec((1, 1, D), lambda i, j: (0, 0, 0)),
            pl.BlockSpec((1, 1, D), lambda i, j: (0, 0, 0)),
        ],
        out_specs=pl.BlockSpec((1, SB, D), lambda i, j: (i, j, 0)),
        out_shape=jax.ShapeDtypeStruct((B, S, D), jnp.float32),
    )(xn, w1, b1.reshape(1, 1, DFF), w2, b2.reshape(1, 1, D),
      g.reshape(1, 1, D), b.reshape(1, 1, D))


def _final_kernel(h_ref, g_ref, b_ref, w_ref, pb_ref, o_ref):
    xn = _ln_rows(h_ref[0], g_ref[0, 0], b_ref[0, 0])
    o_ref[0] = jnp.dot(xn, w_ref[...], preferred_element_type=jnp.float32) + pb_ref[0, 0]


def _final_proj(h, g, b, w, pb):
    SB = 512
    return pl.pallas_call(
        _final_kernel,
        grid=(B, S // SB),
        in_specs=[
            pl.BlockSpec((1, SB, D), lambda i, j: (i, j, 0)),
            pl.BlockSpec((1, 1, D), lambda i, j: (0, 0, 0)),
            pl.BlockSpec((1, 1, D), lambda i, j: (0, 0, 0)),
            pl.BlockSpec((D, NY), lambda i, j: (0, 0)),
            pl.BlockSpec((1, 1, NY), lambda i, j: (0, 0, 0)),
        ],
        out_specs=pl.BlockSpec((1, SB, NY), lambda i, j: (i, j, 0)),
        out_shape=jax.ShapeDtypeStruct((B, S, NY), jnp.float32),
    )(h, g.reshape(1, 1, D), b.reshape(1, 1, D), w, pb.reshape(1, 1, NY))


def _sort_gather(buckets, qkv):
    pos = jnp.arange(S, dtype=jnp.int32)
    comb = buckets * S + pos[None, None, None, :]
    sidx = jnp.argsort(comb, axis=-1)
    sq = jnp.take_along_axis(qkv[None], sidx[..., None], axis=3)
    sqkv = sq.reshape(W, S, 2 * DH)
    sidx_w = sidx.reshape(W, S)
    blocks = sqkv.reshape(W, NBLK, KB, 2 * DH)
    tails = jnp.roll(blocks[:, :, KB - BUCKET:KB, :], 1, axis=1)
    tails = tails.reshape(W * NBLK, BUCKET, 2 * DH)
    return sqkv, tails, sidx_w


def _scatter_back(souts, slse, sidx_w):
    lse_w = slse.reshape(W, S)
    inv = jnp.argsort(sidx_w, axis=-1)
    outs = jnp.take_along_axis(souts, inv[..., None], axis=1)
    lses = jnp.take_along_axis(lse_w, inv, axis=1)
    outs = outs.reshape(NH, B, H, S, DH)
    lses = lses.reshape(NH, B, H, S).transpose(0, 1, 3, 2)
    return outs, lses


def kernel(x, conv_w, temb_w, W_qk, W_v, W_o, b_o, ln1_g, ln1_b, ln2_g, ln2_b,
           ff1_w, ff1_b, ff2_w, ff2_b, enc_norm_g, enc_norm_b, proj_w, proj_b):
    rot = jax.random.normal(jax.random.key(42), (NH, DH, NROT), dtype=jnp.float32)
    x_enc = x.transpose(1, 0, 2)
    xp = jnp.concatenate([x_enc[:, -1:], x_enc, x_enc[:, :1]], axis=1)
    pe = _pos_emb(T, D)
    h = _front(xp, conv_w, pe)
    for l in range(L):
        qkv, buckets = _qkv_buckets(h, W_qk[l], W_v[l], rot)
        sqkv, tails, sidx_w = _sort_gather(buckets.reshape(NH, B, H, S), qkv)
        souts, slse = _attention(sqkv, tails)
        outs, lses = _scatter_back(souts, slse, sidx_w)
        xn = _combine(outs, lses, h, W_o[l], b_o[l], ln1_g[l], ln1_b[l])
        h = _ffn(xn, ff1_w[l], ff1_b[l], ff2_w[l], ff2_b[l], ln2_g[l], ln2_b[l])
    out = _final_proj(h, enc_norm_g, enc_norm_b, proj_w, proj_b)
    return out.transpose(1, 0, 2)
